# baseline (speedup 1.0000x reference)
import functools

import jax
import jax.numpy as jnp
from jax import lax
from jax.experimental import pallas as pl
from jax.experimental.pallas import tpu as pltpu
from jax.experimental.pallas import tpu_sc as plsc

N = 32768
H = 128
B = 16
DAPP = 32
NCLS = 2

NC = 2
NS = 16
NW = NC * NS
RW = N // NW
CH = 128
NCH = RW // CH
NBUF = 4
L = 16


NR = B + 1


def _seg_sum_body(h_hbm, seg_hbm, tgt_hbm, out_hbm,
                  seg_v, tgt_v, acc_v, buf_v, red_v, row_v,
                  acc_sh, gsem):
    c = lax.axis_index("c")
    s = lax.axis_index("s")
    wid = s * NC + c
    base = wid * RW

    pltpu.sync_copy(seg_hbm.at[pl.ds(base, RW)], seg_v)
    pltpu.sync_copy(tgt_hbm.at[pl.ds(base, RW)], tgt_v)

    for i in range(NBUF):
        pltpu.async_copy(h_hbm.at[pl.ds(base + i * CH, CH)],
                         buf_v.at[i], gsem)

    zv = jnp.zeros((L,), jnp.float32)

    def zslot(i, carry):
        acc_v[i // (H // L), pl.ds((i % (H // L)) * L, L)] = zv
        return carry

    lax.fori_loop(0, NR * (H // L), zslot, 0)

    lanes = lax.iota(jnp.int32, L)

    def accumulate(chunk, slot):
        def grp_body(gb, carry):
            g0 = gb * L
            sv = seg_v[pl.ds(chunk * CH + g0, L)]
            tv = tgt_v[pl.ds(chunk * CH + g0, L)]
            rows = jnp.where(tv == 1, sv, B)
            for r in range(L):
                row = rows[r]
                for j in range(H // L):
                    x = buf_v[slot, g0 + r, pl.ds(j * L, L)]
                    plsc.addupdate(acc_v.at[row, pl.ds(j * L, L)], x)
            return carry

        lax.fori_loop(0, CH // L, grp_body, 0)

    for i in range(NCH):
        pltpu.make_async_copy(h_hbm.at[pl.ds(base + i * CH, CH)],
                              buf_v.at[i % NBUF], gsem).wait()
        accumulate(i, i % NBUF)
        if i + NBUF < NCH:
            pltpu.async_copy(h_hbm.at[pl.ds(base + (i + NBUF) * CH, CH)],
                             buf_v.at[i % NBUF], gsem)

    pltpu.sync_copy(acc_v, acc_sh.at[s])
    plsc.subcore_barrier()

    for r in range(NR):
        @pl.when(s == (r % NS))
        def _reduce(r=r):
            for t in range(NS):
                pltpu.sync_copy(acc_sh.at[t, r], red_v.at[t])
            for j in range(H // L):
                acc = red_v[0, pl.ds(j * L, L)]
                for t in range(1, NS):
                    acc = acc + red_v[t, pl.ds(j * L, L)]
                row_v[0, pl.ds(j * L, L)] = acc
            pltpu.sync_copy(row_v, out_hbm.at[c, pl.ds(r, 1)])


@functools.lru_cache(maxsize=1)
def _seg_sum():
    return pl.kernel(
        _seg_sum_body,
        out_type=jax.ShapeDtypeStruct((NC, B + 1, H), jnp.float32),
        mesh=plsc.VectorSubcoreMesh(core_axis_name="c", subcore_axis_name="s"),
        scratch_types=[
            pltpu.VMEM((RW,), jnp.int32),
            pltpu.VMEM((RW,), jnp.int32),
            pltpu.VMEM((NR, H), jnp.float32),
            pltpu.VMEM((NBUF, CH, H), jnp.float32),
            pltpu.VMEM((NS, H), jnp.float32),
            pltpu.VMEM((1, H), jnp.float32),
            pltpu.VMEM_SHARED((NS, NR, H), jnp.float32),
            pltpu.SemaphoreType.DMA,
        ],
    )


def _finalize_body(parts_ref, seg_ref, tgt_ref, dep_ref, feat_ref,
                   w1_ref, w2_ref, w3_ref, b_ref, out_ref):
    gh = parts_ref[0, :B, :] + parts_ref[1, :B, :]
    seg = seg_ref[...]
    tgt = tgt_ref[...]
    dep = dep_ref[...]
    gid = lax.broadcasted_iota(jnp.int32, (B,) + seg.shape, 0)
    m = seg[None, :, :] == gid
    num_tot = jnp.sum(m.astype(jnp.float32), axis=(1, 2))
    num_tgt = jnp.sum(jnp.where(jnp.logical_and(m, tgt[None, :, :] == 1),
                                1.0, 0.0), axis=(1, 2))
    mx = jnp.max(jnp.where(m, dep[None, :, :], -jnp.inf), axis=(1, 2))
    logits = (
        jnp.dot(gh, w1_ref[...], preferred_element_type=jnp.float32)
        + jnp.dot(feat_ref[...], w2_ref[...], preferred_element_type=jnp.float32)
        + mx[:, None] * w3_ref[0, :][None, :]
        + num_tgt[:, None] * w3_ref[1, :][None, :]
        + num_tot[:, None] * w3_ref[2, :][None, :]
        + b_ref[0, :][None, :]
    )
    out_ref[...] = logits


def kernel(h, segment_ids, is_target, depth, feature, W, b):
    seg = segment_ids.astype(jnp.int32)
    tgt = is_target.astype(jnp.int32)
    parts = _seg_sum()(h, seg, tgt)
    logits = pl.pallas_call(
        _finalize_body,
        out_shape=jax.ShapeDtypeStruct((B, NCLS), jnp.float32),
    )(parts, seg.reshape(N // H, H), tgt.reshape(N // H, H),
      depth.reshape(N // H, H), feature,
      W[:H], W[H:H + DAPP], W[H + DAPP:], b.reshape(1, NCLS))
    return logits


# speedup vs baseline: 1.2081x; 1.2081x over previous
import functools

import jax
import jax.numpy as jnp
from jax import lax
from jax.experimental import pallas as pl
from jax.experimental.pallas import tpu as pltpu
from jax.experimental.pallas import tpu_sc as plsc

N = 32768
H = 128
B = 16
DAPP = 32
NCLS = 2

NC = 2
NS = 16
NW = NC * NS
RW = N // NW
CH = 256
NCH = RW // CH
NBUF = 2
L = 16


NR = B + 1


def _seg_sum_body(h_hbm, seg_hbm, tgt_hbm, out_hbm,
                  seg_v, tgt_v, acc_v, buf_v, red_v, row_v,
                  acc_sh, gsem):
    c = lax.axis_index("c")
    s = lax.axis_index("s")
    wid = s * NC + c
    base = wid * RW

    pltpu.sync_copy(seg_hbm.at[pl.ds(base, RW)], seg_v)
    pltpu.sync_copy(tgt_hbm.at[pl.ds(base, RW)], tgt_v)

    for i in range(NBUF):
        pltpu.async_copy(h_hbm.at[pl.ds(base + i * CH, CH)],
                         buf_v.at[i], gsem)

    zv = jnp.zeros((L,), jnp.float32)

    def zslot(i, carry):
        acc_v[i // (H // L), pl.ds((i % (H // L)) * L, L)] = zv
        return carry

    lax.fori_loop(0, NR * (H // L), zslot, 0)

    lanes = lax.iota(jnp.int32, L)

    def accumulate(chunk, slot):
        @plsc.parallel_loop(0, CH // L, 1, unroll=1)
        def grp_body(gb):
            g0 = gb * L
            sv = seg_v[pl.ds(chunk * CH + g0, L)]
            tv = tgt_v[pl.ds(chunk * CH + g0, L)]
            rows = jnp.where(tv == 1, sv, B)
            for r in range(L):
                row = rows[r]
                for j in range(H // L):
                    x = buf_v[slot, g0 + r, pl.ds(j * L, L)]
                    plsc.addupdate(acc_v.at[row, pl.ds(j * L, L)], x)

    for i in range(NCH):
        pltpu.make_async_copy(h_hbm.at[pl.ds(base + i * CH, CH)],
                              buf_v.at[i % NBUF], gsem).wait()
        accumulate(i, i % NBUF)
        if i + NBUF < NCH:
            pltpu.async_copy(h_hbm.at[pl.ds(base + (i + NBUF) * CH, CH)],
                             buf_v.at[i % NBUF], gsem)

    pltpu.sync_copy(acc_v, acc_sh.at[s])
    plsc.subcore_barrier()

    for r in range(NR):
        @pl.when(s == (r % NS))
        def _reduce(r=r):
            for t in range(NS):
                pltpu.sync_copy(acc_sh.at[t, r], red_v.at[t])
            for j in range(H // L):
                acc = red_v[0, pl.ds(j * L, L)]
                for t in range(1, NS):
                    acc = acc + red_v[t, pl.ds(j * L, L)]
                row_v[0, pl.ds(j * L, L)] = acc
            pltpu.sync_copy(row_v, out_hbm.at[c, pl.ds(r, 1)])


@functools.lru_cache(maxsize=1)
def _seg_sum():
    return pl.kernel(
        _seg_sum_body,
        out_type=jax.ShapeDtypeStruct((NC, B + 1, H), jnp.float32),
        mesh=plsc.VectorSubcoreMesh(core_axis_name="c", subcore_axis_name="s"),
        scratch_types=[
            pltpu.VMEM((RW,), jnp.int32),
            pltpu.VMEM((RW,), jnp.int32),
            pltpu.VMEM((NR, H), jnp.float32),
            pltpu.VMEM((NBUF, CH, H), jnp.float32),
            pltpu.VMEM((NS, H), jnp.float32),
            pltpu.VMEM((1, H), jnp.float32),
            pltpu.VMEM_SHARED((NS, NR, H), jnp.float32),
            pltpu.SemaphoreType.DMA,
        ],
    )


def _finalize_body(parts_ref, seg_ref, tgt_ref, dep_ref, feat_ref,
                   w1_ref, w2_ref, w3_ref, b_ref, out_ref):
    gh = parts_ref[0, :B, :] + parts_ref[1, :B, :]
    seg = seg_ref[...]
    tgt = tgt_ref[...]
    dep = dep_ref[...]
    gid = lax.broadcasted_iota(jnp.int32, (B,) + seg.shape, 0)
    m = seg[None, :, :] == gid
    num_tot = jnp.sum(m.astype(jnp.float32), axis=(1, 2))
    num_tgt = jnp.sum(jnp.where(jnp.logical_and(m, tgt[None, :, :] == 1),
                                1.0, 0.0), axis=(1, 2))
    mx = jnp.max(jnp.where(m, dep[None, :, :], -jnp.inf), axis=(1, 2))
    logits = (
        jnp.dot(gh, w1_ref[...], preferred_element_type=jnp.float32)
        + jnp.dot(feat_ref[...], w2_ref[...], preferred_element_type=jnp.float32)
        + mx[:, None] * w3_ref[0, :][None, :]
        + num_tgt[:, None] * w3_ref[1, :][None, :]
        + num_tot[:, None] * w3_ref[2, :][None, :]
        + b_ref[0, :][None, :]
    )
    out_ref[...] = logits


def kernel(h, segment_ids, is_target, depth, feature, W, b):
    seg = segment_ids.astype(jnp.int32)
    tgt = is_target.astype(jnp.int32)
    parts = _seg_sum()(h, seg, tgt)
    logits = pl.pallas_call(
        _finalize_body,
        out_shape=jax.ShapeDtypeStruct((B, NCLS), jnp.float32),
    )(parts, seg.reshape(N // H, H), tgt.reshape(N // H, H),
      depth.reshape(N // H, H), feature,
      W[:H], W[H:H + DAPP], W[H + DAPP:], b.reshape(1, NCLS))
    return logits


# speedup vs baseline: 1.7021x; 1.4089x over previous
import functools

import jax
import jax.numpy as jnp
from jax import lax
from jax.experimental import pallas as pl
from jax.experimental.pallas import tpu as pltpu
from jax.experimental.pallas import tpu_sc as plsc

N = 32768
H = 128
B = 16
DAPP = 32
NCLS = 2

NC = 2
NS = 16
NW = NC * NS
RW = N // NW
CH = 128
NCH = RW // CH
NBUF = 4
L = 16


NR = B + 1


def _seg_sum_body(h_hbm, seg_hbm, tgt_hbm, out_hbm,
                  seg_v, tgt_v, csg2_v, buf_v, zero_v,
                  acc_sh, gsem):
    c = lax.axis_index("c")
    s = lax.axis_index("s")
    wid = s * NC + c
    base = wid * RW

    pltpu.sync_copy(seg_hbm.at[pl.ds(base, RW)], seg_v)
    pltpu.sync_copy(tgt_hbm.at[pl.ds(base, RW)], tgt_v)

    zv = jnp.zeros((L,), jnp.float32)
    lanes = lax.iota(jnp.int32, L)

    @pl.when(s == 0)
    def _zero():
        def zrow(i, carry):
            zero_v[i // (H // L), pl.ds((i % (H // L)) * L, L)] = zv
            return carry

        lax.fori_loop(0, NR * (H // L), zrow, 0)
        pltpu.sync_copy(zero_v, acc_sh)

    trash = jnp.zeros((L,), jnp.int32) + B

    def mkidx(i, carry):
        sv = seg_v[pl.ds(i * L, L)]
        tv = tgt_v[pl.ds(i * L, L)]
        csg2_v[i // (CH // L), pl.ds((i % (CH // L)) * L, L)] = jnp.where(
            tv == 1, sv, trash)
        return carry

    lax.fori_loop(0, RW // L, mkidx, 0)

    plsc.subcore_barrier()

    pltpu.async_copy(h_hbm.at[pl.ds(base, CH)], buf_v.at[0], gsem)
    pltpu.async_copy(h_hbm.at[pl.ds(base + CH, CH)], buf_v.at[1], gsem)

    def chunk_body(i, carry):
        slot = lax.rem(i, 2)
        pltpu.make_async_copy(h_hbm.at[pl.ds(base + i * CH, CH)],
                              buf_v.at[slot], gsem).wait()
        pltpu.sync_copy(buf_v.at[slot], acc_sh.at[csg2_v.at[i]], add=True)

        @pl.when(i + 2 < NCH)
        def _next():
            pltpu.async_copy(h_hbm.at[pl.ds(base + (i + 2) * CH, CH)],
                             buf_v.at[slot], gsem)

        return carry

    lax.fori_loop(0, NCH, chunk_body, 0)

    plsc.subcore_barrier()

    @pl.when(s == 0)
    def _emit():
        pltpu.sync_copy(acc_sh, out_hbm.at[c])


@functools.lru_cache(maxsize=1)
def _seg_sum():
    return pl.kernel(
        _seg_sum_body,
        out_type=jax.ShapeDtypeStruct((NC, B + 1, H), jnp.float32),
        mesh=plsc.VectorSubcoreMesh(core_axis_name="c", subcore_axis_name="s"),
        scratch_types=[
            pltpu.VMEM((RW,), jnp.int32),
            pltpu.VMEM((RW,), jnp.int32),
            pltpu.VMEM((NCH, CH), jnp.int32),
            pltpu.VMEM((2, CH, H), jnp.float32),
            pltpu.VMEM((NR, H), jnp.float32),
            pltpu.VMEM_SHARED((NR, H), jnp.float32),
            pltpu.SemaphoreType.DMA,
        ],
    )


def _finalize_body(parts_ref, seg_ref, tgt_ref, dep_ref, feat_ref,
                   w1_ref, w2_ref, w3_ref, b_ref, out_ref):
    gh = parts_ref[0, :B, :] + parts_ref[1, :B, :]
    seg = seg_ref[...]
    tgt = tgt_ref[...]
    dep = dep_ref[...]
    gid = lax.broadcasted_iota(jnp.int32, (B,) + seg.shape, 0)
    m = seg[None, :, :] == gid
    num_tot = jnp.sum(m.astype(jnp.float32), axis=(1, 2))
    num_tgt = jnp.sum(jnp.where(jnp.logical_and(m, tgt[None, :, :] == 1),
                                1.0, 0.0), axis=(1, 2))
    mx = jnp.max(jnp.where(m, dep[None, :, :], -jnp.inf), axis=(1, 2))
    logits = (
        jnp.dot(gh, w1_ref[...], preferred_element_type=jnp.float32)
        + jnp.dot(feat_ref[...], w2_ref[...], preferred_element_type=jnp.float32)
        + mx[:, None] * w3_ref[0, :][None, :]
        + num_tgt[:, None] * w3_ref[1, :][None, :]
        + num_tot[:, None] * w3_ref[2, :][None, :]
        + b_ref[0, :][None, :]
    )
    out_ref[...] = logits


def kernel(h, segment_ids, is_target, depth, feature, W, b):
    seg = segment_ids.astype(jnp.int32)
    tgt = is_target.astype(jnp.int32)
    parts = _seg_sum()(h, seg, tgt)
    logits = pl.pallas_call(
        _finalize_body,
        out_shape=jax.ShapeDtypeStruct((B, NCLS), jnp.float32),
    )(parts, seg.reshape(N // H, H), tgt.reshape(N // H, H),
      depth.reshape(N // H, H), feature,
      W[:H], W[H:H + DAPP], W[H + DAPP:], b.reshape(1, NCLS))
    return logits
